# baseline (speedup 1.0000x reference)
import functools

import jax
import jax.numpy as jnp
from jax import lax
from jax.experimental import pallas as pl
from jax.experimental.pallas import tpu as pltpu
from jax.experimental.pallas import tpu_sc as plsc

BATCH = 16384
DIM = 32
CHUNK = 128


def _sc_gather_gmf(user2d, item2d, gmf_user, gmf_item, mlp_user, mlp_item):
    info = plsc.get_sparse_core_info()
    nc, ns = info.num_cores, info.num_subcores
    nw = nc * ns
    b_per_w = BATCH // nw
    rows_per_w = b_per_w // CHUNK

    mesh = plsc.VectorSubcoreMesh(core_axis_name="c", subcore_axis_name="s")
    out_sds = jax.ShapeDtypeStruct((BATCH, DIM), jnp.float32)

    @functools.partial(
        pl.kernel,
        mesh=mesh,
        out_type=[out_sds, out_sds, out_sds],
        compiler_params=pltpu.CompilerParams(use_tc_tiling_on_sc=False),
        scratch_types=[
            pltpu.VMEM((rows_per_w, CHUNK), jnp.int32),
            pltpu.VMEM((rows_per_w, CHUNK), jnp.int32),
            pltpu.VMEM((b_per_w, DIM), jnp.float32),
            pltpu.VMEM((b_per_w, DIM), jnp.float32),
            pltpu.VMEM((b_per_w, DIM), jnp.float32),
            pltpu.VMEM((b_per_w, DIM), jnp.float32),
            pltpu.SemaphoreType.DMA,
        ],
    )
    def body(user_h, item_h, gu_h, gi_h, mu_h, mi_h,
             out_gmf, out_mu, out_mi,
             idx_u, idx_v, buf_gu, buf_gi, buf_mu, buf_mi, sem):
        wid = lax.axis_index("s") * nc + lax.axis_index("c")
        row0 = wid * rows_per_w
        base = wid * b_per_w

        pltpu.sync_copy(user_h.at[pl.ds(row0, rows_per_w)], idx_u)
        pltpu.sync_copy(item_h.at[pl.ds(row0, rows_per_w)], idx_v)

        copies = []
        for j in range(rows_per_w):
            sl = pl.ds(j * CHUNK, CHUNK)
            copies.append(pltpu.async_copy(gu_h.at[idx_u.at[j]], buf_gu.at[sl], sem))
            copies.append(pltpu.async_copy(gi_h.at[idx_v.at[j]], buf_gi.at[sl], sem))
            copies.append(pltpu.async_copy(mu_h.at[idx_u.at[j]], buf_mu.at[sl], sem))
            copies.append(pltpu.async_copy(mi_h.at[idx_v.at[j]], buf_mi.at[sl], sem))
        for c in copies:
            c.wait()

        pltpu.sync_copy(buf_mu, out_mu.at[pl.ds(base, b_per_w)])
        pltpu.sync_copy(buf_mi, out_mi.at[pl.ds(base, b_per_w)])

        def prod_row(i, carry):
            a0 = buf_gu[i, pl.ds(0, 16)]
            a1 = buf_gu[i, pl.ds(16, 16)]
            c0 = buf_gi[i, pl.ds(0, 16)]
            c1 = buf_gi[i, pl.ds(16, 16)]
            buf_gu[i, pl.ds(0, 16)] = a0 * c0
            buf_gu[i, pl.ds(16, 16)] = a1 * c1
            return carry

        lax.fori_loop(0, b_per_w, prod_row, 0)
        pltpu.sync_copy(buf_gu, out_gmf.at[pl.ds(base, b_per_w)])

    return body(user2d, item2d, gmf_user, gmf_item, mlp_user, mlp_item)


def _tc_mlp_body(gmf, xu, xi, w1a, w1b, b1, w2, b2, w3, b3, wg, wm, bfc, out):
    h = xu[:] @ w1a[:] + xi[:] @ w1b[:] + b1[:]
    h = jnp.maximum(h, 0.0)
    h = jnp.maximum(h @ w2[:] + b2[:], 0.0)
    h = jnp.maximum(h @ w3[:] + b3[:], 0.0)
    logit = (jnp.sum(gmf[:] * wg[:], axis=1, keepdims=True)
             + jnp.sum(h * wm[:], axis=1, keepdims=True) + bfc[:])
    out[:] = jax.nn.sigmoid(logit)


def _tc_mlp(gmf_prod, mlp_u, mlp_i, W1, b1, W2, b2, W3, b3, Wfc, bfc):
    blk = 2048
    grid = BATCH // blk
    data_spec = pl.BlockSpec((blk, DIM), lambda i: (i, 0))

    def whole(shape):
        return pl.BlockSpec(shape, lambda i: (0, 0))

    w1a = W1[:DIM]
    w1b = W1[DIM:]
    wg = Wfc[:DIM].reshape(1, DIM)
    wm = Wfc[DIM:].reshape(1, 8)

    out = pl.pallas_call(
        _tc_mlp_body,
        grid=(grid,),
        in_specs=[
            data_spec, data_spec, data_spec,
            whole((DIM, 32)), whole((DIM, 32)), whole((1, 32)),
            whole((32, 16)), whole((1, 16)),
            whole((16, 8)), whole((1, 8)),
            whole((1, DIM)), whole((1, 8)), whole((1, 1)),
        ],
        out_specs=pl.BlockSpec((blk, 1), lambda i: (i, 0)),
        out_shape=jax.ShapeDtypeStruct((BATCH, 1), jnp.float32),
    )(gmf_prod, mlp_u, mlp_i,
      w1a, w1b, b1.reshape(1, 32),
      W2, b2.reshape(1, 16),
      W3, b3.reshape(1, 8),
      wg, wm, bfc.reshape(1, 1))
    return out.reshape(BATCH)


def kernel(user, item, gmf_user, gmf_item, mlp_user, mlp_item,
           W1, b1, W2, b2, W3, b3, Wfc, bfc):
    user2d = user.astype(jnp.int32).reshape(BATCH // CHUNK, CHUNK)
    item2d = item.astype(jnp.int32).reshape(BATCH // CHUNK, CHUNK)
    gmf_prod, mlp_u, mlp_i = _sc_gather_gmf(
        user2d, item2d, gmf_user, gmf_item, mlp_user, mlp_item)
    return _tc_mlp(gmf_prod, mlp_u, mlp_i, W1, b1, W2, b2, W3, b3, Wfc, bfc)
